# Initial kernel scaffold; baseline (speedup 1.0000x reference)
import jax
import jax.numpy as jnp
from jax.experimental import pallas as pl


def kernel(indices, attributes):
    raise NotImplementedError("write your pallas kernel here")


# speedup vs baseline: 1.4737x; 1.4737x over previous
import functools

import jax
import jax.numpy as jnp
from jax import lax
from jax.experimental import pallas as pl
from jax.experimental.pallas import tpu as pltpu
from jax.experimental.pallas import tpu_sc as plsc

NUM_CLASSES = 100000
EMBED_DIM = 128
BATCH = 16384

NC, NS, L = 2, 16, 16
NW = NC * NS
B_PER_W = BATCH // NW
CHUNK = 128
N_CHUNK = B_PER_W // CHUNK
D_VECS = EMBED_DIM // L
UNROLL = 4


def _body(idx_hbm, table_hbm, out_hbm, idx_v, rows_v, red_v, sem):
    wid = lax.axis_index("s") * NC + lax.axis_index("c")

    pltpu.sync_copy(idx_hbm.at[pl.ds(wid * N_CHUNK, N_CHUNK)], idx_v)

    copies = [
        pltpu.async_copy(
            table_hbm.at[idx_v.at[j]],
            rows_v.at[pl.ds(j * CHUNK, CHUNK)],
            sem,
        )
        for j in range(N_CHUNK)
    ]
    for c in copies:
        c.wait()

    red_i = red_v.bitcast(jnp.int32)

    def norm_rows(i, carry):
        r0 = i * UNROLL
        accs = []
        for u in range(UNROLL):
            acc = None
            for j in range(D_VECS):
                v = rows_v[r0 + u, pl.ds(j * L, L)]
                sq = v * v
                acc = sq if acc is None else acc + sq
            red_v[u, pl.ds(0, L)] = acc
            red_v[u, pl.ds(L, L)] = acc
            accs.append(acc)
        for d in (8, 4, 2):
            for u in range(UNROLL):
                accs[u] = accs[u] + red_v[u, pl.ds(L - d, L)]
                red_v[u, pl.ds(0, L)] = accs[u]
                red_v[u, pl.ds(L, L)] = accs[u]
        sss = []
        for u in range(UNROLL):
            sss.append(jnp.maximum(accs[u] + red_v[u, pl.ds(L - 1, L)], 1e-24))
        invs = []
        for u in range(UNROLL):
            ss = sss[u]
            y = jnp.full((L,), 64.0, jnp.float32)
            for _ in range(8):
                y = 0.5 * (y + ss / y)
            invs.append(1.0 / y)
        for u in range(UNROLL):
            for j in range(D_VECS):
                r = r0 + u
                rows_v[r, pl.ds(j * L, L)] = (
                    rows_v[r, pl.ds(j * L, L)] * invs[u]
                )
        return carry

    lax.fori_loop(0, B_PER_W // UNROLL, norm_rows, 0)

    pltpu.sync_copy(rows_v, out_hbm.at[pl.ds(wid * B_PER_W, B_PER_W)])


@jax.jit
def _gather_normalize(idx2d, attributes):
    mesh = plsc.VectorSubcoreMesh(core_axis_name="c", subcore_axis_name="s")
    return pl.kernel(
        _body,
        out_type=jax.ShapeDtypeStruct((BATCH, EMBED_DIM), jnp.float32),
        mesh=mesh,
        compiler_params=pltpu.CompilerParams(use_tc_tiling_on_sc=False),
        scratch_types=[
            pltpu.VMEM((N_CHUNK, CHUNK), jnp.int32),
            pltpu.VMEM((B_PER_W, EMBED_DIM), jnp.float32),
            pltpu.VMEM((UNROLL, 2 * L), jnp.float32),
            pltpu.SemaphoreType.DMA,
        ],
    )(idx2d, attributes)


def kernel(indices, attributes):
    idx2d = indices.astype(jnp.int32).reshape(BATCH // CHUNK, CHUNK)
    return _gather_normalize(idx2d, attributes)


# speedup vs baseline: 2.5104x; 1.7035x over previous
import functools

import jax
import jax.numpy as jnp
from jax import lax
from jax.experimental import pallas as pl
from jax.experimental.pallas import tpu as pltpu
from jax.experimental.pallas import tpu_sc as plsc

NUM_CLASSES = 100000
EMBED_DIM = 128
BATCH = 16384

NC, NS, L = 2, 16, 16
NW = NC * NS
B_PER_W = BATCH // NW
CHUNK = 128
N_CHUNK = B_PER_W // CHUNK
D_VECS = EMBED_DIM // L
UNROLL = 4


def _body(idx_hbm, table_hbm, out_hbm, idx_v, rows_v, red_v, sem):
    wid = lax.axis_index("s") * NC + lax.axis_index("c")

    pltpu.sync_copy(idx_hbm.at[pl.ds(wid * N_CHUNK, N_CHUNK)], idx_v)

    copies = [
        pltpu.async_copy(
            table_hbm.at[idx_v.at[j]],
            rows_v.at[pl.ds(j * CHUNK, CHUNK)],
            sem,
        )
        for j in range(N_CHUNK)
    ]
    for c in copies:
        c.wait()

    def norm_rows(i, carry):
        r0 = i * UNROLL
        accs = []
        for u in range(UNROLL):
            acc = None
            for j in range(D_VECS):
                v = rows_v[r0 + u, pl.ds(j * L, L)]
                sq = v * v
                acc = sq if acc is None else acc + sq
            red_v[u, pl.ds(0, L)] = acc
            red_v[u, pl.ds(L, L)] = acc
            accs.append(acc)
        for d in (8, 4, 2):
            for u in range(UNROLL):
                accs[u] = accs[u] + red_v[u, pl.ds(L - d, L)]
                red_v[u, pl.ds(0, L)] = accs[u]
                red_v[u, pl.ds(L, L)] = accs[u]
        sss = []
        for u in range(UNROLL):
            sss.append(jnp.maximum(accs[u] + red_v[u, pl.ds(L - 1, L)], 1e-24))
        invs = []
        for u in range(UNROLL):
            ss = sss[u]
            z = 1.0 / (0.044194174 * ss + 5.6568542)
            hs = 0.5 * ss
            for _ in range(5):
                z = z * (1.5 - hs * z * z)
            invs.append(z)
        for u in range(UNROLL):
            for j in range(D_VECS):
                r = r0 + u
                rows_v[r, pl.ds(j * L, L)] = (
                    rows_v[r, pl.ds(j * L, L)] * invs[u]
                )
        return carry

    lax.fori_loop(0, B_PER_W // UNROLL, norm_rows, 0)

    pltpu.sync_copy(rows_v, out_hbm.at[pl.ds(wid * B_PER_W, B_PER_W)])


@jax.jit
def _gather_normalize(idx2d, attributes):
    mesh = plsc.VectorSubcoreMesh(core_axis_name="c", subcore_axis_name="s")
    return pl.kernel(
        _body,
        out_type=jax.ShapeDtypeStruct((BATCH, EMBED_DIM), jnp.float32),
        mesh=mesh,
        compiler_params=pltpu.CompilerParams(use_tc_tiling_on_sc=False),
        scratch_types=[
            pltpu.VMEM((N_CHUNK, CHUNK), jnp.int32),
            pltpu.VMEM((B_PER_W, EMBED_DIM), jnp.float32),
            pltpu.VMEM((UNROLL, 2 * L), jnp.float32),
            pltpu.SemaphoreType.DMA,
        ],
    )(idx2d, attributes)


def kernel(indices, attributes):
    idx2d = indices.astype(jnp.int32).reshape(BATCH // CHUNK, CHUNK)
    return _gather_normalize(idx2d, attributes)


# speedup vs baseline: 2.5356x; 1.0101x over previous
import functools

import jax
import jax.numpy as jnp
from jax import lax
from jax.experimental import pallas as pl
from jax.experimental.pallas import tpu as pltpu
from jax.experimental.pallas import tpu_sc as plsc

NUM_CLASSES = 100000
EMBED_DIM = 128
BATCH = 16384

NC, NS, L = 2, 16, 16
NW = NC * NS
B_PER_W = BATCH // NW
CHUNK = 128
N_CHUNK = B_PER_W // CHUNK
D_VECS = EMBED_DIM // L
UNROLL = 4


def _body(idx_hbm, table_hbm, out_hbm, idx_v, rows_v, red_v, gsems, wsem):
    wid = lax.axis_index("s") * NC + lax.axis_index("c")

    pltpu.sync_copy(idx_hbm.at[pl.ds(wid * N_CHUNK, N_CHUNK)], idx_v)

    copies = [
        pltpu.async_copy(
            table_hbm.at[idx_v.at[j]],
            rows_v.at[pl.ds(j * CHUNK, CHUNK)],
            gsems[j],
        )
        for j in range(N_CHUNK)
    ]
    writes = []

    def norm_rows(i, carry):
        r0 = i * UNROLL
        accs = []
        for u in range(UNROLL):
            acc = None
            for j in range(D_VECS):
                v = rows_v[r0 + u, pl.ds(j * L, L)]
                sq = v * v
                acc = sq if acc is None else acc + sq
            red_v[u, pl.ds(0, L)] = acc
            red_v[u, pl.ds(L, L)] = acc
            accs.append(acc)
        for d in (8, 4, 2):
            for u in range(UNROLL):
                accs[u] = accs[u] + red_v[u, pl.ds(L - d, L)]
                red_v[u, pl.ds(0, L)] = accs[u]
                red_v[u, pl.ds(L, L)] = accs[u]
        sss = []
        for u in range(UNROLL):
            sss.append(jnp.maximum(accs[u] + red_v[u, pl.ds(L - 1, L)], 1e-24))
        invs = []
        for u in range(UNROLL):
            ss = sss[u]
            z = 1.0 / (0.044194174 * ss + 5.6568542)
            hs = 0.5 * ss
            for _ in range(5):
                z = z * (1.5 - hs * z * z)
            invs.append(z)
        for u in range(UNROLL):
            for j in range(D_VECS):
                r = r0 + u
                rows_v[r, pl.ds(j * L, L)] = (
                    rows_v[r, pl.ds(j * L, L)] * invs[u]
                )
        return carry

    for j in range(N_CHUNK):
        copies[j].wait()
        lax.fori_loop(j * (CHUNK // UNROLL), (j + 1) * (CHUNK // UNROLL),
                      norm_rows, 0)
        writes.append(
            pltpu.async_copy(
                rows_v.at[pl.ds(j * CHUNK, CHUNK)],
                out_hbm.at[pl.ds(wid * B_PER_W + j * CHUNK, CHUNK)],
                wsem,
            )
        )
    for w in writes:
        w.wait()


@jax.jit
def _gather_normalize(idx2d, attributes):
    mesh = plsc.VectorSubcoreMesh(core_axis_name="c", subcore_axis_name="s")
    return pl.kernel(
        _body,
        out_type=jax.ShapeDtypeStruct((BATCH, EMBED_DIM), jnp.float32),
        mesh=mesh,
        compiler_params=pltpu.CompilerParams(use_tc_tiling_on_sc=False),
        scratch_types=[
            pltpu.VMEM((N_CHUNK, CHUNK), jnp.int32),
            pltpu.VMEM((B_PER_W, EMBED_DIM), jnp.float32),
            pltpu.VMEM((UNROLL, 2 * L), jnp.float32),
            [pltpu.SemaphoreType.DMA] * N_CHUNK,
            pltpu.SemaphoreType.DMA,
        ],
    )(idx2d, attributes)


def kernel(indices, attributes):
    idx2d = indices.astype(jnp.int32).reshape(BATCH // CHUNK, CHUNK)
    return _gather_normalize(idx2d, attributes)


# speedup vs baseline: 2.6083x; 1.0287x over previous
import functools

import jax
import jax.numpy as jnp
from jax import lax
from jax.experimental import pallas as pl
from jax.experimental.pallas import tpu as pltpu
from jax.experimental.pallas import tpu_sc as plsc

NUM_CLASSES = 100000
EMBED_DIM = 128
BATCH = 16384

NC, NS, L = 2, 16, 16
NW = NC * NS
B_PER_W = BATCH // NW
CHUNK = 128
N_CHUNK = B_PER_W // CHUNK
D_VECS = EMBED_DIM // L
UNROLL = 8


def _body(idx_hbm, table_hbm, out_hbm, idx_v, rows_v, red_v, gsems, wsem):
    wid = lax.axis_index("s") * NC + lax.axis_index("c")

    pltpu.sync_copy(idx_hbm.at[pl.ds(wid * N_CHUNK, N_CHUNK)], idx_v)

    copies = [
        pltpu.async_copy(
            table_hbm.at[idx_v.at[j]],
            rows_v.at[pl.ds(j * CHUNK, CHUNK)],
            gsems[j],
        )
        for j in range(N_CHUNK)
    ]
    writes = []

    def norm_rows(i, carry):
        r0 = i * UNROLL
        accs = []
        for u in range(UNROLL):
            acc = None
            for j in range(D_VECS):
                v = rows_v[r0 + u, pl.ds(j * L, L)]
                sq = v * v
                acc = sq if acc is None else acc + sq
            red_v[u, pl.ds(0, L)] = acc
            red_v[u, pl.ds(L, L)] = acc
            accs.append(acc)
        for d in (8, 4, 2):
            for u in range(UNROLL):
                accs[u] = accs[u] + red_v[u, pl.ds(L - d, L)]
                red_v[u, pl.ds(0, L)] = accs[u]
                red_v[u, pl.ds(L, L)] = accs[u]
        sss = []
        for u in range(UNROLL):
            sss.append(jnp.maximum(accs[u] + red_v[u, pl.ds(L - 1, L)], 1e-24))
        invs = []
        for u in range(UNROLL):
            ss = sss[u]
            z = 1.0 / (0.044194174 * ss + 5.6568542)
            hs = 0.5 * ss
            for _ in range(5):
                z = z * (1.5 - hs * z * z)
            invs.append(z)
        for u in range(UNROLL):
            for j in range(D_VECS):
                r = r0 + u
                rows_v[r, pl.ds(j * L, L)] = (
                    rows_v[r, pl.ds(j * L, L)] * invs[u]
                )
        return carry

    for j in range(N_CHUNK):
        copies[j].wait()
        lax.fori_loop(j * (CHUNK // UNROLL), (j + 1) * (CHUNK // UNROLL),
                      norm_rows, 0)
        writes.append(
            pltpu.async_copy(
                rows_v.at[pl.ds(j * CHUNK, CHUNK)],
                out_hbm.at[pl.ds(wid * B_PER_W + j * CHUNK, CHUNK)],
                wsem,
            )
        )
    for w in writes:
        w.wait()


@jax.jit
def _gather_normalize(idx2d, attributes):
    mesh = plsc.VectorSubcoreMesh(core_axis_name="c", subcore_axis_name="s")
    return pl.kernel(
        _body,
        out_type=jax.ShapeDtypeStruct((BATCH, EMBED_DIM), jnp.float32),
        mesh=mesh,
        compiler_params=pltpu.CompilerParams(use_tc_tiling_on_sc=False),
        scratch_types=[
            pltpu.VMEM((N_CHUNK, CHUNK), jnp.int32),
            pltpu.VMEM((B_PER_W, EMBED_DIM), jnp.float32),
            pltpu.VMEM((UNROLL, 2 * L), jnp.float32),
            [pltpu.SemaphoreType.DMA] * N_CHUNK,
            pltpu.SemaphoreType.DMA,
        ],
    )(idx2d, attributes)


def kernel(indices, attributes):
    idx2d = indices.astype(jnp.int32).reshape(BATCH // CHUNK, CHUNK)
    return _gather_normalize(idx2d, attributes)
